# baseline (speedup 1.0000x reference)
import jax
import jax.numpy as jnp
from jax.experimental import pallas as pl

_VOCAB = 100000
_DIM = 64
_N = _VOCAB * _DIM
_BLOCK = _N // 10


def _copy_body(x_ref, o_ref):
    o_ref[...] = x_ref[...]


def kernel(embeddings):
    flat = pl.pallas_call(
        _copy_body,
        grid=(_N // _BLOCK,),
        in_specs=[pl.BlockSpec((_BLOCK,), lambda i: (i,))],
        out_specs=pl.BlockSpec((_BLOCK,), lambda i: (i,)),
        out_shape=jax.ShapeDtypeStruct((_N,), jnp.float32),
    )(embeddings.reshape(_N))
    return flat.reshape(_VOCAB, _DIM)
